# baseline (speedup 1.0000x reference)
import jax
import jax.numpy as jnp
from jax import lax
from jax.experimental import pallas as pl
from jax.experimental.pallas import tpu as pltpu
from jax.experimental.pallas import tpu_sc as plsc

_NC = 2
_NS = 16
_L = 16
_NW = _NC * _NS

_N = 4194304
_CHUNK = _N // _NW
_S = 8192
_NSUB = _CHUNK // _S
_BINS = 1024
_SHIFT1 = 21
_SHIFT2 = 11
_U = 4

_mesh = plsc.VectorSubcoreMesh(core_axis_name="c", subcore_axis_name="s")

_OUT = jax.ShapeDtypeStruct((_NW * 2 * _BINS,), jnp.float32)

_NBUF = 3

_SCRATCH = (
    [pltpu.VMEM((_S,), jnp.float32) for _ in range(2 * _NBUF)]
    + [
        pltpu.VMEM((_L * _BINS,), jnp.float32),
        pltpu.VMEM((_L * _BINS,), jnp.float32),
        pltpu.VMEM((2 * _BINS,), jnp.float32),
    ]
    + [pltpu.SemaphoreType.DMA for _ in range(2 * _NBUF)]
)


def _histogram_pass(o_hbm, t_hbm, bvec, out_hbm,
                    o0, o1, o2, t0, t1, t2, hc, hs, stage,
                    so0, so1, so2, st0, st1, st2, second):
    wid = lax.axis_index("s") * _NC + lax.axis_index("c")
    base = wid * _CHUNK

    zeros = jnp.zeros((_L,), jnp.float32)
    ones = jnp.ones((_L,), jnp.float32)
    lane = lax.iota(jnp.int32, _L)
    gath = lax.iota(jnp.int32, _L) * _L

    obufs, tbufs = (o0, o1, o2), (t0, t1, t2)
    osems, tsems = (so0, so1, so2), (st0, st1, st2)

    def start(j):
        off = base + j * _S
        s = j % _NBUF
        co = pltpu.async_copy(o_hbm.at[pl.ds(off, _S)], obufs[s], osems[s])
        ct = pltpu.async_copy(t_hbm.at[pl.ds(off, _S)], tbufs[s], tsems[s])
        return co, ct

    pend = {0: start(0), 1: start(1)}

    def zero_body(v, c):
        off = pl.multiple_of(v * (_L * _L), _L)
        for h in range(_L):
            hc[pl.ds(off + h * _L, _L)] = zeros
            hs[pl.ds(off + h * _L, _L)] = zeros
        return c

    lax.fori_loop(0, _BINS // _L, zero_body, 0)

    def compute(ob, tb):
        @plsc.parallel_loop(0, _S, _L, unroll=_U)
        def body(i):
            off = pl.multiple_of(i, _L)
            o = ob[pl.ds(off, _L)]
            t = tb[pl.ds(off, _L)]
            r = (t - o) / t
            e = r * r
            u = lax.bitcast_convert_type(e, jnp.int32)
            if second:
                coarse = jnp.bitwise_and(
                    lax.shift_right_logical(u, _SHIFT1), _BINS - 1)
                sub = jnp.bitwise_and(
                    lax.shift_right_logical(u, _SHIFT2), _BINS - 1)
                m = coarse == bvec
                idx = lax.shift_left(sub, 4) + lane
                plsc.addupdate_scatter(hc, [idx], ones, mask=m)
                plsc.addupdate_scatter(hs, [idx], e, mask=m)
            else:
                coarse = jnp.bitwise_and(
                    lax.shift_right_logical(u, _SHIFT1), _BINS - 1)
                idx = lax.shift_left(coarse, 4) + lane
                plsc.addupdate_scatter(hc, [idx], ones)
                plsc.addupdate_scatter(hs, [idx], e)

    for j in range(_NSUB):
        if j + 2 < _NSUB:
            pend[j + 2] = start(j + 2)
        co, ct = pend.pop(j)
        co.wait()
        ct.wait()
        compute(obufs[j % _NBUF], tbufs[j % _NBUF])

    def red_body(v, c):
        off = pl.multiple_of(v * _L, _L)
        base = v * (_L * _L)
        acc_c = plsc.load_gather(hc, [gath + base])
        acc_s = plsc.load_gather(hs, [gath + base])
        for h in range(1, _L):
            acc_c = acc_c + plsc.load_gather(hc, [gath + (base + h)])
            acc_s = acc_s + plsc.load_gather(hs, [gath + (base + h)])
        stage[pl.ds(off, _L)] = acc_c
        stage[pl.ds(_BINS + off, _L)] = acc_s
        return c

    lax.fori_loop(0, _BINS // _L, red_body, 0)

    pltpu.sync_copy(stage, out_hbm.at[pl.ds(wid * 2 * _BINS, 2 * _BINS)])


def _pass1_body(o_hbm, t_hbm, out_hbm, *scr):
    _histogram_pass(o_hbm, t_hbm, None, out_hbm, *scr, second=False)


def _pass2_body(o_hbm, t_hbm, b_hbm, out_hbm, *scr):
    bbuf, sb = scr[-2:]
    pltpu.async_copy(b_hbm, bbuf, sb).wait()
    bvec = bbuf[...]
    _histogram_pass(o_hbm, t_hbm, bvec, out_hbm, *scr[:-2], second=True)


_params = pltpu.CompilerParams(needs_layout_passes=False)

_pass1 = pl.kernel(_pass1_body, out_type=_OUT, mesh=_mesh,
                   scratch_types=list(_SCRATCH), compiler_params=_params)
_pass2 = pl.kernel(_pass2_body, out_type=_OUT, mesh=_mesh,
                   scratch_types=list(_SCRATCH)
                   + [pltpu.VMEM((_L,), jnp.int32), pltpu.SemaphoreType.DMA],
                   compiler_params=_params)


def _locate(tot, want):
    cum = jnp.cumsum(tot, axis=1)
    idx = jnp.argmax(cum[0] >= want)
    return idx, cum[0, idx] - tot[0, idx], cum[1, idx] - tot[1, idx]


def kernel(output, target):
    n = output.shape[0]
    k = int(n * 0.97)

    h1 = _pass1(output, target).reshape(_NW, 2, _BINS)
    tot1 = h1.sum(axis=0)
    kf = jnp.float32(k)
    b, below_c, below_s = _locate(tot1, kf)

    bvec = jnp.full((_L,), b.astype(jnp.int32), dtype=jnp.int32)
    h2 = _pass2(output, target, bvec).reshape(_NW, 2, _BINS)
    tot2 = h2.sum(axis=0)
    rem = kf - below_c
    t2, below_c2, below_s2 = _locate(tot2, rem)

    taken = rem - below_c2
    mean_t2 = tot2[1][t2] / jnp.maximum(tot2[0][t2], 1.0)
    total = below_s + below_s2 + taken * mean_t2
    return total / kf


# speedup vs baseline: 1.0267x; 1.0267x over previous
import jax
import jax.numpy as jnp
from jax import lax
from jax.experimental import pallas as pl
from jax.experimental.pallas import tpu as pltpu
from jax.experimental.pallas import tpu_sc as plsc

_NC = 2
_NS = 16
_L = 16
_NW = _NC * _NS

_N = 4194304
_CHUNK = _N // _NW
_S = 16384
_NSUB = _CHUNK // _S
_BINS = 1024
_SHIFT1 = 21
_SHIFT2 = 11
_U = 4

_mesh = plsc.VectorSubcoreMesh(core_axis_name="c", subcore_axis_name="s")

_OUT = jax.ShapeDtypeStruct((_NW * 2 * _BINS,), jnp.float32)

_NBUF = 2

_SCRATCH = (
    [pltpu.VMEM((_S,), jnp.float32) for _ in range(2 * _NBUF)]
    + [
        pltpu.VMEM((_L * _BINS,), jnp.float32),
        pltpu.VMEM((_L * _BINS,), jnp.float32),
        pltpu.VMEM((2 * _BINS,), jnp.float32),
    ]
    + [pltpu.SemaphoreType.DMA for _ in range(2 * _NBUF)]
)


def _histogram_pass(o_hbm, t_hbm, bvec, out_hbm,
                    o0, o1, t0, t1, hc, hs, stage,
                    so0, so1, st0, st1, second):
    wid = lax.axis_index("s") * _NC + lax.axis_index("c")
    base = wid * _CHUNK

    zeros = jnp.zeros((_L,), jnp.float32)
    ones = jnp.ones((_L,), jnp.float32)
    lane = lax.iota(jnp.int32, _L)
    gath = lax.iota(jnp.int32, _L) * _L

    obufs, tbufs = (o0, o1), (t0, t1)
    osems, tsems = (so0, so1), (st0, st1)

    def start(j):
        off = base + j * _S
        s = j % _NBUF
        co = pltpu.async_copy(o_hbm.at[pl.ds(off, _S)], obufs[s], osems[s])
        ct = pltpu.async_copy(t_hbm.at[pl.ds(off, _S)], tbufs[s], tsems[s])
        return co, ct

    pend = {0: start(0)}

    def zero_body(v, c):
        off = pl.multiple_of(v * (_L * _L), _L)
        for h in range(_L):
            hc[pl.ds(off + h * _L, _L)] = zeros
            hs[pl.ds(off + h * _L, _L)] = zeros
        return c

    lax.fori_loop(0, _BINS // _L, zero_body, 0)

    def compute(ob, tb):
        @plsc.parallel_loop(0, _S, _L, unroll=_U)
        def body(i):
            off = pl.multiple_of(i, _L)
            o = ob[pl.ds(off, _L)]
            t = tb[pl.ds(off, _L)]
            r = (t - o) / t
            e = r * r
            u = lax.bitcast_convert_type(e, jnp.int32)
            if second:
                coarse = jnp.bitwise_and(
                    lax.shift_right_logical(u, _SHIFT1), _BINS - 1)
                sub = jnp.bitwise_and(
                    lax.shift_right_logical(u, _SHIFT2), _BINS - 1)
                m = coarse == bvec
                idx = lax.shift_left(sub, 4) + lane
                plsc.addupdate_scatter(hc, [idx], ones, mask=m)
                plsc.addupdate_scatter(hs, [idx], e, mask=m)
            else:
                coarse = jnp.bitwise_and(
                    lax.shift_right_logical(u, _SHIFT1), _BINS - 1)
                idx = lax.shift_left(coarse, 4) + lane
                plsc.addupdate_scatter(hc, [idx], ones)
                plsc.addupdate_scatter(hs, [idx], e)

    for j in range(_NSUB):
        if j + 1 < _NSUB:
            pend[j + 1] = start(j + 1)
        co, ct = pend.pop(j)
        co.wait()
        ct.wait()
        compute(obufs[j % _NBUF], tbufs[j % _NBUF])

    def red_body(v, c):
        off = pl.multiple_of(v * _L, _L)
        base = v * (_L * _L)
        acc_c = plsc.load_gather(hc, [gath + base])
        acc_s = plsc.load_gather(hs, [gath + base])
        for h in range(1, _L):
            acc_c = acc_c + plsc.load_gather(hc, [gath + (base + h)])
            acc_s = acc_s + plsc.load_gather(hs, [gath + (base + h)])
        stage[pl.ds(off, _L)] = acc_c
        stage[pl.ds(_BINS + off, _L)] = acc_s
        return c

    lax.fori_loop(0, _BINS // _L, red_body, 0)

    pltpu.sync_copy(stage, out_hbm.at[pl.ds(wid * 2 * _BINS, 2 * _BINS)])


def _pass1_body(o_hbm, t_hbm, out_hbm, *scr):
    _histogram_pass(o_hbm, t_hbm, None, out_hbm, *scr, second=False)


def _pass2_body(o_hbm, t_hbm, b_hbm, out_hbm, *scr):
    bbuf, sb = scr[-2:]
    pltpu.async_copy(b_hbm, bbuf, sb).wait()
    bvec = bbuf[...]
    _histogram_pass(o_hbm, t_hbm, bvec, out_hbm, *scr[:-2], second=True)


_params = pltpu.CompilerParams(needs_layout_passes=False)

_pass1 = pl.kernel(_pass1_body, out_type=_OUT, mesh=_mesh,
                   scratch_types=list(_SCRATCH), compiler_params=_params)
_pass2 = pl.kernel(_pass2_body, out_type=_OUT, mesh=_mesh,
                   scratch_types=list(_SCRATCH)
                   + [pltpu.VMEM((_L,), jnp.int32), pltpu.SemaphoreType.DMA],
                   compiler_params=_params)


def _locate(tot, want):
    cum = jnp.cumsum(tot, axis=1)
    idx = jnp.argmax(cum[0] >= want)
    return idx, cum[0, idx] - tot[0, idx], cum[1, idx] - tot[1, idx]


def kernel(output, target):
    n = output.shape[0]
    k = int(n * 0.97)

    h1 = _pass1(output, target).reshape(_NW, 2, _BINS)
    tot1 = h1.sum(axis=0)
    kf = jnp.float32(k)
    b, below_c, below_s = _locate(tot1, kf)

    bvec = jnp.full((_L,), b.astype(jnp.int32), dtype=jnp.int32)
    h2 = _pass2(output, target, bvec).reshape(_NW, 2, _BINS)
    tot2 = h2.sum(axis=0)
    rem = kf - below_c
    t2, below_c2, below_s2 = _locate(tot2, rem)

    taken = rem - below_c2
    mean_t2 = tot2[1][t2] / jnp.maximum(tot2[0][t2], 1.0)
    total = below_s + below_s2 + taken * mean_t2
    return total / kf


# speedup vs baseline: 1.0944x; 1.0660x over previous
import jax
import jax.numpy as jnp
from jax import lax
from jax.experimental import pallas as pl
from jax.experimental.pallas import tpu as pltpu
from jax.experimental.pallas import tpu_sc as plsc

_NC = 2
_NS = 16
_L = 16
_NW = _NC * _NS

_N = 4194304
_CHUNK = _N // _NW
_S = 16384
_NSUB = _CHUNK // _S
_BINS = 1024
_SHIFT1 = 21
_SHIFT2 = 11
_U = 4

_mesh = plsc.VectorSubcoreMesh(core_axis_name="c", subcore_axis_name="s")

_OUT = jax.ShapeDtypeStruct((_NW * 2 * _BINS,), jnp.float32)

_NBUF = 2

_SCRATCH = (
    [pltpu.VMEM((_S,), jnp.float32) for _ in range(2 * _NBUF)]
    + [
        pltpu.VMEM((_L * _BINS,), jnp.float32),
        pltpu.VMEM((_L * _BINS,), jnp.float32),
        pltpu.VMEM((2 * _BINS,), jnp.float32),
    ]
    + [pltpu.SemaphoreType.DMA for _ in range(2 * _NBUF)]
)


def _histogram_pass(o_hbm, t_hbm, bvec, out_hbm,
                    o0, o1, t0, t1, hc, hs, stage,
                    so0, so1, st0, st1, second):
    wid = lax.axis_index("s") * _NC + lax.axis_index("c")
    base = wid * _CHUNK

    zeros = jnp.zeros((_L,), jnp.float32)
    ones = jnp.ones((_L,), jnp.float32)
    lane = lax.iota(jnp.int32, _L)
    gath = lax.iota(jnp.int32, _L) * _L

    obufs, tbufs = (o0, o1), (t0, t1)
    osems, tsems = (so0, so1), (st0, st1)

    def start(j):
        off = base + j * _S
        s = j % _NBUF
        co = pltpu.async_copy(o_hbm.at[pl.ds(off, _S)], obufs[s], osems[s])
        ct = pltpu.async_copy(t_hbm.at[pl.ds(off, _S)], tbufs[s], tsems[s])
        return co, ct

    pend = {0: start(0)}

    def zero_body(v, c):
        off = pl.multiple_of(v * (_L * _L), _L)
        for h in range(_L):
            hc[pl.ds(off + h * _L, _L)] = zeros
            hs[pl.ds(off + h * _L, _L)] = zeros
        return c

    lax.fori_loop(0, _BINS // _L, zero_body, 0)

    def compute(ob, tb):
        @plsc.parallel_loop(0, _S, _L, unroll=_U)
        def body(i):
            off = pl.multiple_of(i, _L)
            o = ob[pl.ds(off, _L)]
            t = tb[pl.ds(off, _L)]
            r = (t - o) / t
            e = r * r
            u = lax.bitcast_convert_type(e, jnp.int32)
            if second:
                coarse = jnp.bitwise_and(
                    lax.shift_right_logical(u, _SHIFT1), _BINS - 1)
                sub = jnp.bitwise_and(
                    lax.shift_right_logical(u, _SHIFT2), _BINS - 1)
                m = coarse == bvec
                idx = lax.shift_left(sub, 4) + lane
                plsc.addupdate_scatter(hc, [idx], ones, mask=m)
                plsc.addupdate_scatter(hs, [idx], e, mask=m)
            else:
                coarse = jnp.bitwise_and(
                    lax.shift_right_logical(u, _SHIFT1), _BINS - 1)
                idx = lax.shift_left(coarse, 4) + lane
                plsc.addupdate_scatter(hc, [idx], ones)
                plsc.addupdate_scatter(hs, [idx], e)

    for j in range(_NSUB):
        if j + 1 < _NSUB:
            pend[j + 1] = start(j + 1)
        co, ct = pend.pop(j)
        co.wait()
        ct.wait()
        compute(obufs[j % _NBUF], tbufs[j % _NBUF])

    def red_body(v, c):
        off = pl.multiple_of(v * _L, _L)
        base = v * (_L * _L)
        acc_c = plsc.load_gather(hc, [gath + base])
        acc_s = plsc.load_gather(hs, [gath + base])
        for h in range(1, _L):
            acc_c = acc_c + plsc.load_gather(hc, [gath + (base + h)])
            acc_s = acc_s + plsc.load_gather(hs, [gath + (base + h)])
        stage[pl.ds(off, _L)] = acc_c
        stage[pl.ds(_BINS + off, _L)] = acc_s
        return c

    lax.fori_loop(0, _BINS // _L, red_body, 0)

    pltpu.sync_copy(stage, out_hbm.at[pl.ds(wid * 2 * _BINS, 2 * _BINS)])


def _pass1_body(o_hbm, t_hbm, out_hbm, *scr):
    _histogram_pass(o_hbm, t_hbm, None, out_hbm, *scr, second=False)


def _pass2_body(o_hbm, t_hbm, b_hbm, out_hbm, *scr):
    bbuf, sb = scr[-2:]
    pltpu.async_copy(b_hbm, bbuf, sb).wait()
    bvec = bbuf[...]
    _histogram_pass(o_hbm, t_hbm, bvec, out_hbm, *scr[:-2], second=True)


_params = pltpu.CompilerParams(needs_layout_passes=False)

_pass1 = pl.kernel(_pass1_body, out_type=_OUT, mesh=_mesh,
                   scratch_types=list(_SCRATCH), compiler_params=_params)
_pass2 = pl.kernel(_pass2_body, out_type=_OUT, mesh=_mesh,
                   scratch_types=list(_SCRATCH)
                   + [pltpu.VMEM((_L,), jnp.int32), pltpu.SemaphoreType.DMA],
                   compiler_params=_params)


def kernel(output, target):
    n = output.shape[0]
    k = int(n * 0.97)
    kf = jnp.float32(k)

    h1 = _pass1(output, target).reshape(_NW, 2, _BINS)
    tot1 = h1.sum(axis=0)
    ccum1 = jnp.cumsum(tot1[0])
    below1 = ccum1 < kf
    b = below1.sum(dtype=jnp.int32)
    bel1 = jnp.sum(jnp.where(below1[None, :], tot1, 0.0), axis=1)
    below_c, below_s = bel1[0], bel1[1]

    bvec = jnp.full((_L,), b, dtype=jnp.int32)
    h2 = _pass2(output, target, bvec).reshape(_NW, 2, _BINS)
    tot2 = h2.sum(axis=0)
    rem = kf - below_c
    ccum2 = jnp.cumsum(tot2[0])
    below2 = ccum2 < rem
    at2 = jnp.logical_and(ccum2 >= rem, ccum2 - tot2[0] < rem)
    bel2 = jnp.sum(jnp.where(below2[None, :], tot2, 0.0), axis=1)
    att2 = jnp.sum(jnp.where(at2[None, :], tot2, 0.0), axis=1)
    below_c2, below_s2 = bel2[0], bel2[1]
    cnt_t2, sum_t2 = att2[0], att2[1]

    taken = rem - below_c2
    mean_t2 = sum_t2 / jnp.maximum(cnt_t2, 1.0)
    total = below_s + below_s2 + taken * mean_t2
    return total / kf


# speedup vs baseline: 1.2650x; 1.1559x over previous
import jax
import jax.numpy as jnp
from jax import lax
from jax.experimental import pallas as pl
from jax.experimental.pallas import tpu as pltpu
from jax.experimental.pallas import tpu_sc as plsc

_NC = 2
_NS = 16
_L = 16
_NW = _NC * _NS

_N = 4194304
_CHUNK = _N // _NW
_S1 = 16384
_NSUB1 = _CHUNK // _S1
_S2 = 32768
_NSUB2 = _CHUNK // _S2
_BINS = 512
_SHIFT1 = 22
_SHIFT2 = 13
_U = 4

_mesh = plsc.VectorSubcoreMesh(core_axis_name="c", subcore_axis_name="s")

_params = pltpu.CompilerParams(needs_layout_passes=False)

_HIST = jax.ShapeDtypeStruct((_NW * 2 * _BINS,), jnp.float32)
_ECACHE = jax.ShapeDtypeStruct((_N,), jnp.float32)

_lane = lambda: lax.iota(jnp.int32, _L)


def _zero_hists(hc, hs):
    zeros = jnp.zeros((_L,), jnp.float32)

    def zero_body(v, c):
        off = pl.multiple_of(v * (_L * _L), _L)
        for h in range(_L):
            hc[pl.ds(off + h * _L, _L)] = zeros
            hs[pl.ds(off + h * _L, _L)] = zeros
        return c

    lax.fori_loop(0, _BINS // _L, zero_body, 0)


def _reduce_and_emit(hc, hs, stage, out_hbm, wid):
    gath = lax.iota(jnp.int32, _L) * _L

    def red_body(v, c):
        off = pl.multiple_of(v * _L, _L)
        base = v * (_L * _L)
        acc_c = plsc.load_gather(hc, [gath + base])
        acc_s = plsc.load_gather(hs, [gath + base])
        for h in range(1, _L):
            acc_c = acc_c + plsc.load_gather(hc, [gath + (base + h)])
            acc_s = acc_s + plsc.load_gather(hs, [gath + (base + h)])
        stage[pl.ds(off, _L)] = acc_c
        stage[pl.ds(_BINS + off, _L)] = acc_s
        return c

    lax.fori_loop(0, _BINS // _L, red_body, 0)

    pltpu.sync_copy(stage, out_hbm.at[pl.ds(wid * 2 * _BINS, 2 * _BINS)])


def _pass1_body(o_hbm, t_hbm, out_hbm, e_hbm,
                o0, o1, t0, t1, e0, e1, hc, hs, stage,
                so0, so1, st0, st1, se0, se1):
    wid = lax.axis_index("s") * _NC + lax.axis_index("c")
    base = wid * _CHUNK

    ones = jnp.ones((_L,), jnp.float32)
    lane = _lane()

    obufs, tbufs, ebufs = (o0, o1), (t0, t1), (e0, e1)
    osems, tsems, esems = (so0, so1), (st0, st1), (se0, se1)

    def start(j):
        off = base + j * _S1
        s = j % 2
        co = pltpu.async_copy(o_hbm.at[pl.ds(off, _S1)], obufs[s], osems[s])
        ct = pltpu.async_copy(t_hbm.at[pl.ds(off, _S1)], tbufs[s], tsems[s])
        return co, ct

    def start_eout(j):
        off = base + j * _S1
        s = j % 2
        return pltpu.async_copy(ebufs[s], e_hbm.at[pl.ds(off, _S1)], esems[s])

    pend = {0: start(0)}
    _zero_hists(hc, hs)

    def compute(ob, tb, eb):
        @plsc.parallel_loop(0, _S1, _L, unroll=_U)
        def body(i):
            off = pl.multiple_of(i, _L)
            o = ob[pl.ds(off, _L)]
            t = tb[pl.ds(off, _L)]
            r = (t - o) / t
            e = r * r
            eb[pl.ds(off, _L)] = e
            u = lax.bitcast_convert_type(e, jnp.int32)
            coarse = jnp.bitwise_and(
                lax.shift_right_logical(u, _SHIFT1), _BINS - 1)
            idx = lax.shift_left(coarse, 4) + lane
            plsc.addupdate_scatter(hc, [idx], ones)
            plsc.addupdate_scatter(hs, [idx], e)

    epend = {}
    for j in range(_NSUB1):
        if j + 1 < _NSUB1:
            pend[j + 1] = start(j + 1)
        co, ct = pend.pop(j)
        co.wait()
        ct.wait()
        if j - 2 in epend:
            epend.pop(j - 2).wait()
        compute(obufs[j % 2], tbufs[j % 2], ebufs[j % 2])
        epend[j] = start_eout(j)
    for j in sorted(epend):
        epend.pop(j).wait()

    _reduce_and_emit(hc, hs, stage, out_hbm, wid)


def _pass2_body(e_hbm, b_hbm, out_hbm,
                e0, e1, hc, hs, stage, se0, se1, bbuf, sb):
    wid = lax.axis_index("s") * _NC + lax.axis_index("c")
    base = wid * _CHUNK

    ones = jnp.ones((_L,), jnp.float32)
    lane = _lane()

    ebufs, esems = (e0, e1), (se0, se1)

    def start(j):
        off = base + j * _S2
        s = j % 2
        return pltpu.async_copy(e_hbm.at[pl.ds(off, _S2)], ebufs[s], esems[s])

    pend = {0: start(0)}
    pltpu.async_copy(b_hbm, bbuf, sb).wait()
    bvec = bbuf[...]
    _zero_hists(hc, hs)

    def compute(eb):
        @plsc.parallel_loop(0, _S2, _L, unroll=_U)
        def body(i):
            off = pl.multiple_of(i, _L)
            e = eb[pl.ds(off, _L)]
            u = lax.bitcast_convert_type(e, jnp.int32)
            coarse = jnp.bitwise_and(
                lax.shift_right_logical(u, _SHIFT1), _BINS - 1)
            sub = jnp.bitwise_and(
                lax.shift_right_logical(u, _SHIFT2), _BINS - 1)
            m = coarse == bvec
            idx = lax.shift_left(sub, 4) + lane
            plsc.addupdate_scatter(hc, [idx], ones, mask=m)
            plsc.addupdate_scatter(hs, [idx], e, mask=m)

    for j in range(_NSUB2):
        if j + 1 < _NSUB2:
            pend[j + 1] = start(j + 1)
        pend.pop(j).wait()
        compute(ebufs[j % 2])

    _reduce_and_emit(hc, hs, stage, out_hbm, wid)


_pass1 = pl.kernel(
    _pass1_body, out_type=(_HIST, _ECACHE), mesh=_mesh,
    scratch_types=(
        [pltpu.VMEM((_S1,), jnp.float32) for _ in range(6)]
        + [pltpu.VMEM((_L * _BINS,), jnp.float32),
           pltpu.VMEM((_L * _BINS,), jnp.float32),
           pltpu.VMEM((2 * _BINS,), jnp.float32)]
        + [pltpu.SemaphoreType.DMA for _ in range(6)]
    ),
    compiler_params=_params)

_pass2 = pl.kernel(
    _pass2_body, out_type=_HIST, mesh=_mesh,
    scratch_types=(
        [pltpu.VMEM((_S2,), jnp.float32) for _ in range(2)]
        + [pltpu.VMEM((_L * _BINS,), jnp.float32),
           pltpu.VMEM((_L * _BINS,), jnp.float32),
           pltpu.VMEM((2 * _BINS,), jnp.float32)]
        + [pltpu.SemaphoreType.DMA for _ in range(2)]
        + [pltpu.VMEM((_L,), jnp.int32), pltpu.SemaphoreType.DMA]
    ),
    compiler_params=_params)


def kernel(output, target):
    n = output.shape[0]
    k = int(n * 0.97)
    kf = jnp.float32(k)

    h1, ecache = _pass1(output, target)
    tot1 = h1.reshape(_NW, 2, _BINS).sum(axis=0)
    ccum1 = jnp.cumsum(tot1[0])
    below1 = ccum1 < kf
    b = below1.sum(dtype=jnp.int32)
    bel1 = jnp.sum(jnp.where(below1[None, :], tot1, 0.0), axis=1)
    below_c, below_s = bel1[0], bel1[1]

    bvec = jnp.full((_L,), b, dtype=jnp.int32)
    h2 = _pass2(ecache, bvec)
    tot2 = h2.reshape(_NW, 2, _BINS).sum(axis=0)
    rem = kf - below_c
    ccum2 = jnp.cumsum(tot2[0])
    below2 = ccum2 < rem
    at2 = jnp.logical_and(ccum2 >= rem, ccum2 - tot2[0] < rem)
    bel2 = jnp.sum(jnp.where(below2[None, :], tot2, 0.0), axis=1)
    att2 = jnp.sum(jnp.where(at2[None, :], tot2, 0.0), axis=1)
    below_c2, below_s2 = bel2[0], bel2[1]
    cnt_t2, sum_t2 = att2[0], att2[1]

    taken = rem - below_c2
    mean_t2 = sum_t2 / jnp.maximum(cnt_t2, 1.0)
    total = below_s + below_s2 + taken * mean_t2
    return total / kf
